# baseline (speedup 1.0000x reference)
import functools

import jax
import jax.numpy as jnp
from jax import lax
from jax.experimental import pallas as pl
from jax.experimental.pallas import tpu as pltpu
from jax.experimental.pallas import tpu_sc as plsc

_NC = 2
_NS = 16
_NW = _NC * _NS

_IW = 128
_K = 4


def _emb_call(n_rows, d):
    rows_per_w = n_rows // _NW
    steps = rows_per_w // _K
    mesh = plsc.VectorSubcoreMesh(core_axis_name="c", subcore_axis_name="s")

    @functools.partial(
        pl.kernel,
        out_type=jax.ShapeDtypeStruct((n_rows * _IW, d), jnp.float32),
        mesh=mesh,
        scratch_types=[
            pltpu.VMEM((_K, _IW), jnp.int32),
            pltpu.VMEM((_K * _IW, d), jnp.float32),
            pltpu.SemaphoreType.DMA,
        ],
        compiler_params=pltpu.CompilerParams(use_tc_tiling_on_sc=False),
    )
    def emb(idx_hbm, w_hbm, out_hbm, idx_v, rows_v, sem):
        wid = lax.axis_index("s") * _NC + lax.axis_index("c")
        row0 = wid * rows_per_w

        def body(c, carry):
            r = row0 + c * _K
            pltpu.sync_copy(idx_hbm.at[pl.ds(r, _K)], idx_v)
            copies = [
                pltpu.async_copy(
                    w_hbm.at[idx_v.at[j]],
                    rows_v.at[pl.ds(j * _IW, _IW)],
                    sem,
                )
                for j in range(_K)
            ]
            for cp in copies:
                cp.wait()
            pltpu.sync_copy(rows_v, out_hbm.at[pl.ds(r * _IW, _K * _IW)])
            return carry

        lax.fori_loop(0, steps, body, 0, unroll=False)

    return emb


def kernel(idx, weight):
    b, s = idx.shape
    v, d = weight.shape
    n = b * s
    idx2 = idx.reshape(n // _IW, _IW).astype(jnp.int32)
    out = _emb_call(n // _IW, d)(idx2, weight)
    return out.reshape(b, s, d)


# speedup vs baseline: 1.0435x; 1.0435x over previous
import functools

import jax
import jax.numpy as jnp
from jax import lax
from jax.experimental import pallas as pl
from jax.experimental.pallas import tpu as pltpu
from jax.experimental.pallas import tpu_sc as plsc

_NC = 2
_NS = 16
_NW = _NC * _NS

_IW = 128
_K = 4
_NBUF = 2


def _emb_call(n_rows, d):
    rows_per_w = n_rows // _NW
    steps = rows_per_w // _K
    groups = steps // _NBUF
    mesh = plsc.VectorSubcoreMesh(core_axis_name="c", subcore_axis_name="s")

    scratch = (
        [pltpu.VMEM((_K, _IW), jnp.int32) for _ in range(_NBUF)]
        + [pltpu.VMEM((_K * _IW, d), jnp.float32) for _ in range(_NBUF)]
        + [pltpu.SemaphoreType.DMA for _ in range(3 * _NBUF)]
    )

    @functools.partial(
        pl.kernel,
        out_type=jax.ShapeDtypeStruct((n_rows * _IW, d), jnp.float32),
        mesh=mesh,
        scratch_types=scratch,
        compiler_params=pltpu.CompilerParams(use_tc_tiling_on_sc=False),
    )
    def emb(idx_hbm, w_hbm, out_hbm, *refs):
        idx_v = refs[:_NBUF]
        rows_v = refs[_NBUF:2 * _NBUF]
        sem_i = refs[2 * _NBUF:3 * _NBUF]
        sem_g = refs[3 * _NBUF:4 * _NBUF]
        sem_s = refs[4 * _NBUF:5 * _NBUF]

        wid = lax.axis_index("s") * _NC + lax.axis_index("c")
        row0 = wid * rows_per_w

        def fire_gathers(b):
            for j in range(_K):
                pltpu.async_copy(
                    w_hbm.at[idx_v[b].at[j]],
                    rows_v[b].at[pl.ds(j * _IW, _IW)],
                    sem_g[b],
                )

        def drain_gathers(b):
            pltpu.make_async_copy(
                w_hbm.at[pl.ds(0, _K * _IW)], rows_v[b], sem_g[b]
            ).wait()

        def wait_idx(b):
            pltpu.make_async_copy(
                idx_hbm.at[pl.ds(0, _K)], idx_v[b], sem_i[b]
            ).wait()

        def wait_store(b):
            pltpu.make_async_copy(
                rows_v[b], out_hbm.at[pl.ds(0, _K * _IW)], sem_s[b]
            ).wait()

        for b in range(_NBUF):
            pltpu.async_copy(idx_hbm.at[pl.ds(row0 + b * _K, _K)], idx_v[b],
                             sem_i[b])

        def body(g, carry):
            for b in range(_NBUF):
                c = g * _NBUF + b
                wait_idx(b)
                @pl.when(g >= 1)
                def _(b=b):
                    wait_store(b)
                fire_gathers(b)

                m_off = c - 1
                bp = (b - 1) % _NBUF

                def retire():
                    drain_gathers(bp)
                    pltpu.async_copy(
                        rows_v[bp],
                        out_hbm.at[pl.ds((row0 + m_off * _K) * _IW,
                                         _K * _IW)],
                        sem_s[bp],
                    )

                def prefetch_idx():
                    pltpu.async_copy(
                        idx_hbm.at[pl.ds(row0 + (m_off + _NBUF) * _K, _K)],
                        idx_v[bp], sem_i[bp])

                if b == 0:
                    @pl.when(g >= 1)
                    def _():
                        retire()
                        prefetch_idx()
                else:
                    retire()
                    @pl.when(m_off + _NBUF < steps)
                    def _():
                        prefetch_idx()
            return carry

        lax.fori_loop(0, groups, body, 0, unroll=False)

        last = steps - 1
        bl = last % _NBUF
        drain_gathers(bl)
        pltpu.async_copy(
            rows_v[bl],
            out_hbm.at[pl.ds((row0 + last * _K) * _IW, _K * _IW)],
            sem_s[bl],
        )
        for b in range(_NBUF):
            wait_store(b)

    return emb


def kernel(idx, weight):
    b, s = idx.shape
    v, d = weight.shape
    n = b * s
    idx2 = idx.reshape(n // _IW, _IW).astype(jnp.int32)
    out = _emb_call(n // _IW, d)(idx2, weight)
    return out.reshape(b, s, d)
